# baseline (speedup 1.0000x reference)
import functools

import jax
import jax.numpy as jnp
from jax import lax
from jax.experimental import pallas as pl
from jax.experimental.pallas import tpu as pltpu
from jax.experimental.pallas import tpu_sc as plsc

_NC = 2
_NS = 16
_NW = _NC * _NS


def _pack2(x):
    h = x.shape[1] // 2
    lo = jax.lax.bitcast_convert_type(
        x[:, :h].astype(jnp.bfloat16), jnp.uint16).astype(jnp.uint32)
    hi = jax.lax.bitcast_convert_type(
        x[:, h:].astype(jnp.bfloat16), jnp.uint16).astype(jnp.uint32)
    return jax.lax.bitcast_convert_type(lo | (hi << 16), jnp.float32)


def _unpack2(x):
    u = jax.lax.bitcast_convert_type(x, jnp.uint32)
    lo = jax.lax.bitcast_convert_type(u << 16, jnp.float32)
    hi = jax.lax.bitcast_convert_type(u & jnp.uint32(0xFFFF0000), jnp.float32)
    return jnp.concatenate([lo, hi], axis=1)


def _proj_body(nf_ref, ws_ref, wd_ref, wv_ref, bv_ref, p_ref, q_ref, v_ref):
    n = nf_ref[...]
    p_ref[...] = _pack2(jnp.dot(n, ws_ref[...], preferred_element_type=jnp.float32))
    q_ref[...] = _pack2(jnp.dot(n, wd_ref[...], preferred_element_type=jnp.float32))
    v_ref[...] = jnp.dot(n, wv_ref[...], preferred_element_type=jnp.float32) + bv_ref[...]


def _node_proj(node_feats, Ws, Wd, Wv, bv, block=2000):
    n_nodes, in_node = node_feats.shape
    hidden = Ws.shape[1]
    out_node = Wv.shape[1]
    grid = (n_nodes // block,)
    return pl.pallas_call(
        _proj_body,
        grid=grid,
        in_specs=[
            pl.BlockSpec((block, in_node), lambda i: (i, 0)),
            pl.BlockSpec((in_node, hidden), lambda i: (0, 0)),
            pl.BlockSpec((in_node, hidden), lambda i: (0, 0)),
            pl.BlockSpec((in_node, out_node), lambda i: (0, 0)),
            pl.BlockSpec((1, out_node), lambda i: (0, 0)),
        ],
        out_specs=[
            pl.BlockSpec((block, hidden // 2), lambda i: (i, 0)),
            pl.BlockSpec((block, hidden // 2), lambda i: (i, 0)),
            pl.BlockSpec((block, out_node), lambda i: (i, 0)),
        ],
        out_shape=[
            jax.ShapeDtypeStruct((n_nodes, hidden // 2), jnp.float32),
            jax.ShapeDtypeStruct((n_nodes, hidden // 2), jnp.float32),
            jax.ShapeDtypeStruct((n_nodes, out_node), jnp.float32),
        ],
    )(node_feats, Ws, Wd, Wv, bv)


def _make_sc_gather(n_edges, hidden, chunk, dt):
    per_w = n_edges // _NW
    n_full = per_w // chunk
    tail = per_w % chunk
    assert tail % 8 == 0
    n_chunks = n_full + (1 if tail else 0)
    nbuf = 4
    n_main = max(0, (n_full - (nbuf - 1))) // nbuf
    n_static = n_chunks - nbuf * n_main
    mesh = plsc.VectorSubcoreMesh(core_axis_name="c", subcore_axis_name="s")

    def _size(c):
        return tail if (tail and c == n_chunks - 1) else chunk

    @functools.partial(
        pl.kernel,
        mesh=mesh,
        compiler_params=pltpu.CompilerParams(use_tc_tiling_on_sc=True),
        out_type=(
            jax.ShapeDtypeStruct((n_edges, hidden), dt),
            jax.ShapeDtypeStruct((n_edges, hidden), dt),
        ),
        scratch_types=[
            pltpu.VMEM((per_w,), jnp.int32),
            pltpu.VMEM((per_w,), jnp.int32),
        ] + [pltpu.VMEM((chunk, hidden), dt)] * (2 * nbuf)
          + [pltpu.SemaphoreType.DMA] * (4 * nbuf),
    )
    def gather2(p_hbm, q_hbm, src_hbm, dst_hbm, gp_hbm, gq_hbm, *scr):
        si_v, di_v = scr[:2]
        prs = scr[2:2 + nbuf]
        qrs = scr[2 + nbuf:2 + 2 * nbuf]
        sems = scr[2 + 2 * nbuf:]
        sgp, sgq = sems[:nbuf], sems[nbuf:2 * nbuf]
        swp, swq = sems[2 * nbuf:3 * nbuf], sems[3 * nbuf:]
        wid = lax.axis_index("s") * _NC + lax.axis_index("c")
        w_base = wid * per_w

        pltpu.sync_copy(src_hbm.at[pl.ds(w_base, per_w)], si_v)
        pltpu.sync_copy(dst_hbm.at[pl.ds(w_base, per_w)], di_v)

        def g_copies(c, b, sz=chunk):
            isl = pl.ds(c * chunk, sz)
            dsl = pl.ds(0, sz)
            return (pltpu.make_async_copy(p_hbm.at[si_v.at[isl]],
                                          prs[b].at[dsl], sgp[b]),
                    pltpu.make_async_copy(q_hbm.at[di_v.at[isl]],
                                          qrs[b].at[dsl], sgq[b]))

        def w_copies(c, b, sz=chunk):
            osl = pl.ds(w_base + c * chunk, sz)
            dsl = pl.ds(0, sz)
            return (pltpu.make_async_copy(prs[b].at[dsl], gp_hbm.at[osl], swp[b]),
                    pltpu.make_async_copy(qrs[b].at[dsl], gq_hbm.at[osl], swq[b]))

        def g_start(c, b, sz=chunk):
            for cp in g_copies(c, b, sz):
                cp.start()

        def process(c, b, static_tail):
            sz = _size(c) if static_tail is not None else chunk
            for cp in g_copies(c, b, sz):
                cp.wait()
            for cp in w_copies(c, b, sz):
                cp.start()
            pb = (b - 1) % nbuf
            if static_tail is None:
                @pl.when(jnp.logical_and(c >= 1, c + nbuf - 1 < n_chunks))
                def _():
                    for cp in w_copies(c - 1, pb):
                        cp.wait()
                    g_start(c + nbuf - 1, pb)
            elif static_tail:
                for cp in w_copies(c - 1, pb, _size(c - 1)):
                    cp.wait()
                g_start(c + nbuf - 1, pb, _size(c + nbuf - 1))

        for b in range(nbuf):
            g_start(b, b, _size(b))

        def body(g, carry):
            for k in range(nbuf):
                process(nbuf * g + k, k, None)
            return carry

        lax.fori_loop(0, n_main, body, 0)

        for c in range(nbuf * n_main, n_chunks):
            process(c, c % nbuf, c >= 1 and c + nbuf - 1 < n_chunks)
        for c in range(n_chunks - nbuf, n_chunks):
            for cp in w_copies(c, c % nbuf, _size(c)):
                cp.wait()

    return gather2


def _edge_body(gp_ref, gq_ref, ef_ref, w1e_ref, b1_ref, w2_ref, b2_ref, out_ref):
    c = jnp.dot(ef_ref[...], w1e_ref[...], preferred_element_type=jnp.float32)
    g = _unpack2(gp_ref[...]) + _unpack2(gq_ref[...])
    h = jnp.maximum(g + c + b1_ref[...], 0.0)
    out_ref[...] = jnp.dot(h, w2_ref[...], preferred_element_type=jnp.float32) + b2_ref[...]


def _edge_mlp(Gp, Gq, edge_feats, W1e, b1, W2, b2, block=4000):
    n_edges, hidden_p = Gp.shape
    in_edge = edge_feats.shape[1]
    out_edge = W2.shape[1]
    grid = (n_edges // block,)
    return pl.pallas_call(
        _edge_body,
        grid=grid,
        in_specs=[
            pl.BlockSpec((block, hidden_p), lambda i: (i, 0)),
            pl.BlockSpec((block, hidden_p), lambda i: (i, 0)),
            pl.BlockSpec((block, in_edge), lambda i: (i, 0)),
            pl.BlockSpec((in_edge, W1e.shape[1]), lambda i: (0, 0)),
            pl.BlockSpec((1, W1e.shape[1]), lambda i: (0, 0)),
            pl.BlockSpec((W2.shape[0], out_edge), lambda i: (0, 0)),
            pl.BlockSpec((1, out_edge), lambda i: (0, 0)),
        ],
        out_specs=pl.BlockSpec((block, out_edge), lambda i: (i, 0)),
        out_shape=jax.ShapeDtypeStruct((n_edges, out_edge), jnp.float32),
    )(Gp, Gq, edge_feats, W1e, b1, W2, b2)


def kernel(node_feats, edge_feats, edge_index, W1, b1, W2, b2, Wv, bv):
    in_node = node_feats.shape[1]
    in_edge = edge_feats.shape[1]
    n_edges = edge_feats.shape[0]
    hidden = W1.shape[1]

    Ws = W1[:in_node]
    We = W1[in_node:in_node + in_edge]
    Wd = W1[in_node + in_edge:]

    P, Q, V = _node_proj(node_feats, Ws, Wd, Wv, bv.reshape(1, -1))

    src = edge_index[0].astype(jnp.int32)
    dst = edge_index[1].astype(jnp.int32)

    Gp, Gq = _make_sc_gather(n_edges, hidden // 2, 96, jnp.float32)(
        P, Q, src, dst)
    e2 = _edge_mlp(Gp, Gq, edge_feats, We, b1.reshape(1, -1), W2,
                   b2.reshape(1, -1))
    return (V, e2)


# speedup vs baseline: 1.4201x; 1.4201x over previous
import functools

import jax
import jax.numpy as jnp
from jax import lax
from jax.experimental import pallas as pl
from jax.experimental.pallas import tpu as pltpu
from jax.experimental.pallas import tpu_sc as plsc

_NC = 2
_NS = 16
_NW = _NC * _NS


def _pack2(x):
    h = x.shape[1] // 2
    lo = jax.lax.bitcast_convert_type(
        x[:, :h].astype(jnp.bfloat16), jnp.uint16).astype(jnp.uint32)
    hi = jax.lax.bitcast_convert_type(
        x[:, h:].astype(jnp.bfloat16), jnp.uint16).astype(jnp.uint32)
    return jax.lax.bitcast_convert_type(lo | (hi << 16), jnp.float32)


def _unpack2(x):
    u = jax.lax.bitcast_convert_type(x, jnp.uint32)
    lo = jax.lax.bitcast_convert_type(u << 16, jnp.float32)
    hi = jax.lax.bitcast_convert_type(u & jnp.uint32(0xFFFF0000), jnp.float32)
    return jnp.concatenate([lo, hi], axis=1)


def _proj_body(nf_ref, ws_ref, wd_ref, wv_ref, bv_ref, p_ref, q_ref, v_ref):
    n = nf_ref[...]
    p_ref[...] = _pack2(jnp.dot(n, ws_ref[...], preferred_element_type=jnp.float32))
    q_ref[...] = _pack2(jnp.dot(n, wd_ref[...], preferred_element_type=jnp.float32))
    v_ref[...] = jnp.dot(n, wv_ref[...], preferred_element_type=jnp.float32) + bv_ref[...]


def _node_proj(node_feats, Ws, Wd, Wv, bv, block=2000):
    n_nodes, in_node = node_feats.shape
    hidden = Ws.shape[1]
    out_node = Wv.shape[1]
    grid = (n_nodes // block,)
    return pl.pallas_call(
        _proj_body,
        grid=grid,
        in_specs=[
            pl.BlockSpec((block, in_node), lambda i: (i, 0)),
            pl.BlockSpec((in_node, hidden), lambda i: (0, 0)),
            pl.BlockSpec((in_node, hidden), lambda i: (0, 0)),
            pl.BlockSpec((in_node, out_node), lambda i: (0, 0)),
            pl.BlockSpec((1, out_node), lambda i: (0, 0)),
        ],
        out_specs=[
            pl.BlockSpec((block, hidden // 2), lambda i: (i, 0)),
            pl.BlockSpec((block, hidden // 2), lambda i: (i, 0)),
            pl.BlockSpec((block, out_node), lambda i: (i, 0)),
        ],
        out_shape=[
            jax.ShapeDtypeStruct((n_nodes, hidden // 2), jnp.float32),
            jax.ShapeDtypeStruct((n_nodes, hidden // 2), jnp.float32),
            jax.ShapeDtypeStruct((n_nodes, out_node), jnp.float32),
        ],
    )(node_feats, Ws, Wd, Wv, bv)


def _make_sc_gather(n_edges, hidden, chunk, dt):
    per_w = n_edges // _NW
    n_full = per_w // chunk
    tail = per_w % chunk
    assert tail % 8 == 0
    n_chunks = n_full + (1 if tail else 0)
    nbuf = 4
    n_main = max(0, (n_full - (nbuf - 1))) // nbuf
    n_static = n_chunks - nbuf * n_main
    mesh = plsc.VectorSubcoreMesh(core_axis_name="c", subcore_axis_name="s")

    def _size(c):
        return tail if (tail and c == n_chunks - 1) else chunk

    @functools.partial(
        pl.kernel,
        mesh=mesh,
        compiler_params=pltpu.CompilerParams(use_tc_tiling_on_sc=True),
        out_type=(
            jax.ShapeDtypeStruct((n_edges, hidden), dt),
            jax.ShapeDtypeStruct((n_edges, hidden), dt),
        ),
        scratch_types=[
            pltpu.VMEM((per_w,), jnp.int32),
            pltpu.VMEM((per_w,), jnp.int32),
        ] + [pltpu.VMEM((chunk, hidden), dt)] * (2 * nbuf)
          + [pltpu.SemaphoreType.DMA] * (4 * nbuf),
    )
    def gather2(p_hbm, q_hbm, src_hbm, dst_hbm, gp_hbm, gq_hbm, *scr):
        si_v, di_v = scr[:2]
        prs = scr[2:2 + nbuf]
        qrs = scr[2 + nbuf:2 + 2 * nbuf]
        sems = scr[2 + 2 * nbuf:]
        sgp, sgq = sems[:nbuf], sems[nbuf:2 * nbuf]
        swp, swq = sems[2 * nbuf:3 * nbuf], sems[3 * nbuf:]
        wid = lax.axis_index("s") * _NC + lax.axis_index("c")
        w_base = wid * per_w

        pltpu.sync_copy(src_hbm.at[pl.ds(w_base, per_w)], si_v)
        pltpu.sync_copy(dst_hbm.at[pl.ds(w_base, per_w)], di_v)

        def g_copies(c, b, sz=chunk):
            isl = pl.ds(c * chunk, sz)
            dsl = pl.ds(0, sz)
            return (pltpu.make_async_copy(p_hbm.at[si_v.at[isl]],
                                          prs[b].at[dsl], sgp[b]),
                    pltpu.make_async_copy(q_hbm.at[di_v.at[isl]],
                                          qrs[b].at[dsl], sgq[b]))

        def w_copies(c, b, sz=chunk):
            osl = pl.ds(w_base + c * chunk, sz)
            dsl = pl.ds(0, sz)
            return (pltpu.make_async_copy(prs[b].at[dsl], gp_hbm.at[osl], swp[b]),
                    pltpu.make_async_copy(qrs[b].at[dsl], gq_hbm.at[osl], swq[b]))

        def g_start(c, b, sz=chunk):
            for cp in g_copies(c, b, sz):
                cp.start()

        def process(c, b, static_tail):
            sz = _size(c) if static_tail is not None else chunk
            for cp in g_copies(c, b, sz):
                cp.wait()
            for cp in w_copies(c, b, sz):
                cp.start()
            pb = (b - 1) % nbuf
            if static_tail is None:
                @pl.when(jnp.logical_and(c >= 1, c + nbuf - 1 < n_chunks))
                def _():
                    for cp in w_copies(c - 1, pb):
                        cp.wait()
                    g_start(c + nbuf - 1, pb)
            elif static_tail:
                for cp in w_copies(c - 1, pb, _size(c - 1)):
                    cp.wait()
                g_start(c + nbuf - 1, pb, _size(c + nbuf - 1))

        for b in range(nbuf):
            g_start(b, b, _size(b))

        def body(g, carry):
            for k in range(nbuf):
                process(nbuf * g + k, k, None)
            return carry

        lax.fori_loop(0, n_main, body, 0)

        for c in range(nbuf * n_main, n_chunks):
            process(c, c % nbuf, c >= 1 and c + nbuf - 1 < n_chunks)
        for c in range(n_chunks - nbuf, n_chunks):
            for cp in w_copies(c, c % nbuf, _size(c)):
                cp.wait()

    return gather2


def _edge_body(gp_ref, gq_ref, eft_ref, w1e_ref, b1_ref, w2_ref, b2t_ref, out_ref):
    c = jax.lax.dot_general(eft_ref[...], w1e_ref[...], (((0,), (0,)), ((), ())),
                            preferred_element_type=jnp.float32)
    g = _unpack2(gp_ref[...]) + _unpack2(gq_ref[...])
    h = jnp.maximum(g + c + b1_ref[...], 0.0)
    out_ref[...] = jax.lax.dot_general(
        w2_ref[...], h, (((0,), (1,)), ((), ())),
        preferred_element_type=jnp.float32) + b2t_ref[...]


def _edge_mlp(Gp, Gq, edge_feats, W1e, b1, W2, b2, block=3200):
    n_edges, hidden_p = Gp.shape
    in_edge = edge_feats.shape[1]
    out_edge = W2.shape[1]
    eft = edge_feats.T
    grid = (n_edges // block,)
    e2t = pl.pallas_call(
        _edge_body,
        grid=grid,
        in_specs=[
            pl.BlockSpec((block, hidden_p), lambda i: (i, 0)),
            pl.BlockSpec((block, hidden_p), lambda i: (i, 0)),
            pl.BlockSpec((in_edge, block), lambda i: (0, i)),
            pl.BlockSpec((in_edge, W1e.shape[1]), lambda i: (0, 0)),
            pl.BlockSpec((1, W1e.shape[1]), lambda i: (0, 0)),
            pl.BlockSpec((W2.shape[0], out_edge), lambda i: (0, 0)),
            pl.BlockSpec((out_edge, 1), lambda i: (0, 0)),
        ],
        out_specs=pl.BlockSpec((out_edge, block), lambda i: (0, i)),
        out_shape=jax.ShapeDtypeStruct((out_edge, n_edges), jnp.float32),
    )(Gp, Gq, eft, W1e, b1, W2, b2.reshape(-1, 1))
    return e2t.T


def kernel(node_feats, edge_feats, edge_index, W1, b1, W2, b2, Wv, bv):
    in_node = node_feats.shape[1]
    in_edge = edge_feats.shape[1]
    n_edges = edge_feats.shape[0]
    hidden = W1.shape[1]

    Ws = W1[:in_node]
    We = W1[in_node:in_node + in_edge]
    Wd = W1[in_node + in_edge:]

    P, Q, V = _node_proj(node_feats, Ws, Wd, Wv, bv.reshape(1, -1))

    src = edge_index[0].astype(jnp.int32)
    dst = edge_index[1].astype(jnp.int32)

    Gp, Gq = _make_sc_gather(n_edges, hidden // 2, 96, jnp.float32)(
        P, Q, src, dst)
    e2 = _edge_mlp(Gp, Gq, edge_feats, We, b1.reshape(1, -1), W2,
                   b2.reshape(1, -1))
    return (V, e2)


# speedup vs baseline: 1.5362x; 1.0817x over previous
import functools

import jax
import jax.numpy as jnp
from jax import lax
from jax.experimental import pallas as pl
from jax.experimental.pallas import tpu as pltpu
from jax.experimental.pallas import tpu_sc as plsc

_NC = 2
_NS = 16
_NW = _NC * _NS


def _pack2(x):
    h = x.shape[1] // 2
    lo = jax.lax.bitcast_convert_type(
        x[:, :h].astype(jnp.bfloat16), jnp.uint16).astype(jnp.uint32)
    hi = jax.lax.bitcast_convert_type(
        x[:, h:].astype(jnp.bfloat16), jnp.uint16).astype(jnp.uint32)
    return jax.lax.bitcast_convert_type(lo | (hi << 16), jnp.float32)


def _unpack2(x):
    u = jax.lax.bitcast_convert_type(x, jnp.uint32)
    lo = jax.lax.bitcast_convert_type(u << 16, jnp.float32)
    hi = jax.lax.bitcast_convert_type(u & jnp.uint32(0xFFFF0000), jnp.float32)
    return jnp.concatenate([lo, hi], axis=1)


def _proj_body(nf_ref, ws_ref, wd_ref, wv_ref, bv_ref, p_ref, q_ref, v_ref):
    n = nf_ref[...]
    p_ref[...] = _pack2(jnp.dot(n, ws_ref[...], preferred_element_type=jnp.float32))
    q_ref[...] = _pack2(jnp.dot(n, wd_ref[...], preferred_element_type=jnp.float32))
    v_ref[...] = jnp.dot(n, wv_ref[...], preferred_element_type=jnp.float32) + bv_ref[...]


def _node_proj(node_feats, Ws, Wd, Wv, bv, block=2000):
    n_nodes, in_node = node_feats.shape
    hidden = Ws.shape[1]
    out_node = Wv.shape[1]
    grid = (n_nodes // block,)
    return pl.pallas_call(
        _proj_body,
        grid=grid,
        in_specs=[
            pl.BlockSpec((block, in_node), lambda i: (i, 0)),
            pl.BlockSpec((in_node, hidden), lambda i: (0, 0)),
            pl.BlockSpec((in_node, hidden), lambda i: (0, 0)),
            pl.BlockSpec((in_node, out_node), lambda i: (0, 0)),
            pl.BlockSpec((1, out_node), lambda i: (0, 0)),
        ],
        out_specs=[
            pl.BlockSpec((block, hidden // 2), lambda i: (i, 0)),
            pl.BlockSpec((block, hidden // 2), lambda i: (i, 0)),
            pl.BlockSpec((block, out_node), lambda i: (i, 0)),
        ],
        out_shape=[
            jax.ShapeDtypeStruct((n_nodes, hidden // 2), jnp.float32),
            jax.ShapeDtypeStruct((n_nodes, hidden // 2), jnp.float32),
            jax.ShapeDtypeStruct((n_nodes, out_node), jnp.float32),
        ],
    )(node_feats, Ws, Wd, Wv, bv)


def _make_sc_gather(n_edges, hidden, chunk, dt):
    per_w = n_edges // _NW
    n_full = per_w // chunk
    tail = per_w % chunk
    assert tail % 8 == 0
    n_chunks = n_full + (1 if tail else 0)
    nbuf = 4
    n_main = max(0, (n_full - (nbuf - 1))) // nbuf
    n_static = n_chunks - nbuf * n_main
    mesh = plsc.VectorSubcoreMesh(core_axis_name="c", subcore_axis_name="s")

    def _size(c):
        return tail if (tail and c == n_chunks - 1) else chunk

    @functools.partial(
        pl.kernel,
        mesh=mesh,
        out_type=(
            jax.ShapeDtypeStruct((n_edges, hidden), dt),
            jax.ShapeDtypeStruct((n_edges, hidden), dt),
        ),
        scratch_types=[
            pltpu.VMEM((per_w,), jnp.int32),
            pltpu.VMEM((per_w,), jnp.int32),
        ] + [pltpu.VMEM((chunk, hidden), dt)] * (2 * nbuf)
          + [pltpu.SemaphoreType.DMA] * (4 * nbuf),
    )
    def gather2(p_hbm, q_hbm, src_hbm, dst_hbm, gp_hbm, gq_hbm, *scr):
        si_v, di_v = scr[:2]
        prs = scr[2:2 + nbuf]
        qrs = scr[2 + nbuf:2 + 2 * nbuf]
        sems = scr[2 + 2 * nbuf:]
        sgp, sgq = sems[:nbuf], sems[nbuf:2 * nbuf]
        swp, swq = sems[2 * nbuf:3 * nbuf], sems[3 * nbuf:]
        wid = lax.axis_index("s") * _NC + lax.axis_index("c")
        w_base = wid * per_w

        pltpu.sync_copy(src_hbm.at[pl.ds(w_base, per_w)], si_v)
        pltpu.sync_copy(dst_hbm.at[pl.ds(w_base, per_w)], di_v)

        def g_copies(c, b, sz=chunk):
            isl = pl.ds(c * chunk, sz)
            dsl = pl.ds(0, sz)
            return (pltpu.make_async_copy(p_hbm.at[si_v.at[isl]],
                                          prs[b].at[dsl], sgp[b]),
                    pltpu.make_async_copy(q_hbm.at[di_v.at[isl]],
                                          qrs[b].at[dsl], sgq[b]))

        def w_copies(c, b, sz=chunk):
            osl = pl.ds(w_base + c * chunk, sz)
            dsl = pl.ds(0, sz)
            return (pltpu.make_async_copy(prs[b].at[dsl], gp_hbm.at[osl], swp[b]),
                    pltpu.make_async_copy(qrs[b].at[dsl], gq_hbm.at[osl], swq[b]))

        def g_start(c, b, sz=chunk):
            for cp in g_copies(c, b, sz):
                cp.start()

        def process(c, b, static_tail):
            sz = _size(c) if static_tail is not None else chunk
            for cp in g_copies(c, b, sz):
                cp.wait()
            for cp in w_copies(c, b, sz):
                cp.start()
            pb = (b - 1) % nbuf
            if static_tail is None:
                @pl.when(jnp.logical_and(c >= 1, c + nbuf - 1 < n_chunks))
                def _():
                    for cp in w_copies(c - 1, pb):
                        cp.wait()
                    g_start(c + nbuf - 1, pb)
            elif static_tail:
                for cp in w_copies(c - 1, pb, _size(c - 1)):
                    cp.wait()
                g_start(c + nbuf - 1, pb, _size(c + nbuf - 1))

        for b in range(nbuf):
            g_start(b, b, _size(b))

        def body(g, carry):
            for k in range(nbuf):
                process(nbuf * g + k, k, None)
            return carry

        lax.fori_loop(0, n_main, body, 0)

        for c in range(nbuf * n_main, n_chunks):
            process(c, c % nbuf, c >= 1 and c + nbuf - 1 < n_chunks)
        for c in range(n_chunks - nbuf, n_chunks):
            for cp in w_copies(c, c % nbuf, _size(c)):
                cp.wait()

    return gather2


def _edge_body(gp_ref, gq_ref, eft_ref, w1e_ref, b1_ref, w2_ref, b2t_ref, out_ref):
    c = jax.lax.dot_general(eft_ref[...], w1e_ref[...], (((0,), (0,)), ((), ())),
                            preferred_element_type=jnp.float32)
    g = _unpack2(gp_ref[...]) + _unpack2(gq_ref[...])
    h = jnp.maximum(g + c + b1_ref[...], 0.0)
    out_ref[...] = jax.lax.dot_general(
        w2_ref[...], h, (((0,), (1,)), ((), ())),
        preferred_element_type=jnp.float32) + b2t_ref[...]


def _edge_mlp(Gp, Gq, edge_feats, W1e, b1, W2, b2, block=6400):
    n_edges, hidden_p = Gp.shape
    in_edge = edge_feats.shape[1]
    out_edge = W2.shape[1]
    eft = edge_feats.T
    grid = (n_edges // block,)
    e2t = pl.pallas_call(
        _edge_body,
        grid=grid,
        in_specs=[
            pl.BlockSpec((block, hidden_p), lambda i: (i, 0)),
            pl.BlockSpec((block, hidden_p), lambda i: (i, 0)),
            pl.BlockSpec((in_edge, block), lambda i: (0, i)),
            pl.BlockSpec((in_edge, W1e.shape[1]), lambda i: (0, 0)),
            pl.BlockSpec((1, W1e.shape[1]), lambda i: (0, 0)),
            pl.BlockSpec((W2.shape[0], out_edge), lambda i: (0, 0)),
            pl.BlockSpec((out_edge, 1), lambda i: (0, 0)),
        ],
        out_specs=pl.BlockSpec((out_edge, block), lambda i: (0, i)),
        out_shape=jax.ShapeDtypeStruct((out_edge, n_edges), jnp.float32),
    )(Gp, Gq, eft, W1e, b1, W2, b2.reshape(-1, 1))
    return e2t.T


def kernel(node_feats, edge_feats, edge_index, W1, b1, W2, b2, Wv, bv):
    in_node = node_feats.shape[1]
    in_edge = edge_feats.shape[1]
    n_edges = edge_feats.shape[0]
    hidden = W1.shape[1]

    Ws = W1[:in_node]
    We = W1[in_node:in_node + in_edge]
    Wd = W1[in_node + in_edge:]

    P, Q, V = _node_proj(node_feats, Ws, Wd, Wv, bv.reshape(1, -1))

    src_i = edge_index[0].astype(jnp.int32)
    dst_i = edge_index[1].astype(jnp.int32)
    Gp, Gq = _make_sc_gather(n_edges, hidden // 2, 96, jnp.float32)(
        P, Q, src_i, dst_i)
    e2 = _edge_mlp(Gp, Gq, edge_feats, We, b1.reshape(1, -1), W2,
                   b2.reshape(1, -1))
    return (V, e2)


# speedup vs baseline: 1.5984x; 1.0405x over previous
import functools

import jax
import jax.numpy as jnp
from jax import lax
from jax.experimental import pallas as pl
from jax.experimental.pallas import tpu as pltpu
from jax.experimental.pallas import tpu_sc as plsc

_NC = 2
_NS = 16
_NW = _NC * _NS


def _pack2(x):
    h = x.shape[1] // 2
    lo = jax.lax.bitcast_convert_type(
        x[:, :h].astype(jnp.bfloat16), jnp.uint16).astype(jnp.uint32)
    hi = jax.lax.bitcast_convert_type(
        x[:, h:].astype(jnp.bfloat16), jnp.uint16).astype(jnp.uint32)
    return jax.lax.bitcast_convert_type(lo | (hi << 16), jnp.float32)


def _unpack2(x):
    u = jax.lax.bitcast_convert_type(x, jnp.uint32)
    lo = jax.lax.bitcast_convert_type(u << 16, jnp.float32)
    hi = jax.lax.bitcast_convert_type(u & jnp.uint32(0xFFFF0000), jnp.float32)
    return jnp.concatenate([lo, hi], axis=1)


def _proj_body(nf_ref, ws_ref, wd_ref, wv_ref, bv_ref, p_ref, q_ref, v_ref):
    n = nf_ref[...]
    p_ref[...] = _pack2(jnp.dot(n, ws_ref[...], preferred_element_type=jnp.float32))
    q_ref[...] = _pack2(jnp.dot(n, wd_ref[...], preferred_element_type=jnp.float32))
    v_ref[...] = jnp.dot(n, wv_ref[...], preferred_element_type=jnp.float32) + bv_ref[...]


def _node_proj(node_feats, Ws, Wd, Wv, bv, block=2000):
    n_nodes, in_node = node_feats.shape
    hidden = Ws.shape[1]
    out_node = Wv.shape[1]
    grid = (n_nodes // block,)
    return pl.pallas_call(
        _proj_body,
        grid=grid,
        in_specs=[
            pl.BlockSpec((block, in_node), lambda i: (i, 0)),
            pl.BlockSpec((in_node, hidden), lambda i: (0, 0)),
            pl.BlockSpec((in_node, hidden), lambda i: (0, 0)),
            pl.BlockSpec((in_node, out_node), lambda i: (0, 0)),
            pl.BlockSpec((1, out_node), lambda i: (0, 0)),
        ],
        out_specs=[
            pl.BlockSpec((block, hidden // 2), lambda i: (i, 0)),
            pl.BlockSpec((block, hidden // 2), lambda i: (i, 0)),
            pl.BlockSpec((block, out_node), lambda i: (i, 0)),
        ],
        out_shape=[
            jax.ShapeDtypeStruct((n_nodes, hidden // 2), jnp.float32),
            jax.ShapeDtypeStruct((n_nodes, hidden // 2), jnp.float32),
            jax.ShapeDtypeStruct((n_nodes, out_node), jnp.float32),
        ],
    )(node_feats, Ws, Wd, Wv, bv)


def _make_sc_gather(n_edges, hidden, chunk, dt):
    per_w = n_edges // _NW
    n_full = per_w // chunk
    tail = per_w % chunk
    assert tail % 8 == 0
    n_chunks = n_full + (1 if tail else 0)
    nbuf = 4
    n_main = max(0, (n_full - (nbuf - 1))) // nbuf
    n_static = n_chunks - nbuf * n_main
    mesh = plsc.VectorSubcoreMesh(core_axis_name="c", subcore_axis_name="s")

    def _size(c):
        return tail if (tail and c == n_chunks - 1) else chunk

    @functools.partial(
        pl.kernel,
        mesh=mesh,
        out_type=(
            jax.ShapeDtypeStruct((n_edges, hidden), dt),
            jax.ShapeDtypeStruct((n_edges, hidden), dt),
        ),
        scratch_types=[
            pltpu.VMEM((per_w,), jnp.int32),
            pltpu.VMEM((per_w,), jnp.int32),
        ] + [pltpu.VMEM((chunk, hidden), dt)] * (2 * nbuf)
          + [pltpu.SemaphoreType.DMA] * (4 * nbuf),
    )
    def gather2(p_hbm, q_hbm, src_hbm, dst_hbm, gp_hbm, gq_hbm, *scr):
        si_v, di_v = scr[:2]
        prs = scr[2:2 + nbuf]
        qrs = scr[2 + nbuf:2 + 2 * nbuf]
        sems = scr[2 + 2 * nbuf:]
        sgp, sgq = sems[:nbuf], sems[nbuf:2 * nbuf]
        swp, swq = sems[2 * nbuf:3 * nbuf], sems[3 * nbuf:]
        wid = lax.axis_index("s") * _NC + lax.axis_index("c")
        w_base = wid * per_w

        pltpu.sync_copy(src_hbm.at[pl.ds(w_base, per_w)], si_v)
        pltpu.sync_copy(dst_hbm.at[pl.ds(w_base, per_w)], di_v)

        def g_copies(c, b, sz=chunk):
            isl = pl.ds(c * chunk, sz)
            dsl = pl.ds(0, sz)
            return (pltpu.make_async_copy(p_hbm.at[si_v.at[isl]],
                                          prs[b].at[dsl], sgp[b]),
                    pltpu.make_async_copy(q_hbm.at[di_v.at[isl]],
                                          qrs[b].at[dsl], sgq[b]))

        def w_copies(c, b, sz=chunk):
            osl = pl.ds(w_base + c * chunk, sz)
            dsl = pl.ds(0, sz)
            return (pltpu.make_async_copy(prs[b].at[dsl], gp_hbm.at[osl], swp[b]),
                    pltpu.make_async_copy(qrs[b].at[dsl], gq_hbm.at[osl], swq[b]))

        def g_start(c, b, sz=chunk):
            for cp in g_copies(c, b, sz):
                cp.start()

        def process(c, b, static_tail):
            sz = _size(c) if static_tail is not None else chunk
            for cp in g_copies(c, b, sz):
                cp.wait()
            for cp in w_copies(c, b, sz):
                cp.start()
            pb = (b - 1) % nbuf
            if static_tail is None:
                @pl.when(jnp.logical_and(c >= 1, c + nbuf - 1 < n_chunks))
                def _():
                    for cp in w_copies(c - 1, pb):
                        cp.wait()
                    g_start(c + nbuf - 1, pb)
            elif static_tail:
                for cp in w_copies(c - 1, pb, _size(c - 1)):
                    cp.wait()
                g_start(c + nbuf - 1, pb, _size(c + nbuf - 1))

        for b in range(nbuf):
            g_start(b, b, _size(b))

        def body(g, carry):
            for k in range(nbuf):
                process(nbuf * g + k, k, None)
            return carry

        lax.fori_loop(0, n_main, body, 0)

        for c in range(nbuf * n_main, n_chunks):
            process(c, c % nbuf, c >= 1 and c + nbuf - 1 < n_chunks)
        for c in range(n_chunks - nbuf, n_chunks):
            for cp in w_copies(c, c % nbuf, _size(c)):
                cp.wait()

    return gather2


def _edge_body(gp_ref, gq_ref, eft_ref, w1e_ref, b1_ref, w2_ref, b2t_ref, out_ref):
    c = jax.lax.dot_general(eft_ref[...], w1e_ref[...], (((0,), (0,)), ((), ())),
                            preferred_element_type=jnp.float32)
    g = _unpack2(gp_ref[...]) + _unpack2(gq_ref[...])
    h = jnp.maximum(g + c + b1_ref[...], 0.0)
    out_ref[...] = jax.lax.dot_general(
        w2_ref[...], h, (((0,), (1,)), ((), ())),
        preferred_element_type=jnp.float32) + b2t_ref[...]


def _edge_mlp(Gp, Gq, edge_feats, W1e, b1, W2, b2, block=12800):
    n_edges, hidden_p = Gp.shape
    in_edge = edge_feats.shape[1]
    out_edge = W2.shape[1]
    eft = edge_feats.T
    grid = (n_edges // block,)
    e2t = pl.pallas_call(
        _edge_body,
        grid=grid,
        in_specs=[
            pl.BlockSpec((block, hidden_p), lambda i: (i, 0)),
            pl.BlockSpec((block, hidden_p), lambda i: (i, 0)),
            pl.BlockSpec((in_edge, block), lambda i: (0, i)),
            pl.BlockSpec((in_edge, W1e.shape[1]), lambda i: (0, 0)),
            pl.BlockSpec((1, W1e.shape[1]), lambda i: (0, 0)),
            pl.BlockSpec((W2.shape[0], out_edge), lambda i: (0, 0)),
            pl.BlockSpec((out_edge, 1), lambda i: (0, 0)),
        ],
        out_specs=pl.BlockSpec((out_edge, block), lambda i: (0, i)),
        out_shape=jax.ShapeDtypeStruct((out_edge, n_edges), jnp.float32),
    )(Gp, Gq, eft, W1e, b1, W2, b2.reshape(-1, 1))
    return e2t.T


def kernel(node_feats, edge_feats, edge_index, W1, b1, W2, b2, Wv, bv):
    in_node = node_feats.shape[1]
    in_edge = edge_feats.shape[1]
    n_edges = edge_feats.shape[0]
    hidden = W1.shape[1]

    Ws = W1[:in_node]
    We = W1[in_node:in_node + in_edge]
    Wd = W1[in_node + in_edge:]

    P, Q, V = _node_proj(node_feats, Ws, Wd, Wv, bv.reshape(1, -1))

    src_i = edge_index[0].astype(jnp.int32)
    dst_i = edge_index[1].astype(jnp.int32)
    Gp, Gq = _make_sc_gather(n_edges, hidden // 2, 96, jnp.float32)(
        P, Q, src_i, dst_i)
    e2 = _edge_mlp(Gp, Gq, edge_feats, We, b1.reshape(1, -1), W2,
                   b2.reshape(1, -1))
    return (V, e2)


# speedup vs baseline: 1.6053x; 1.0043x over previous
import functools

import jax
import jax.numpy as jnp
from jax import lax
from jax.experimental import pallas as pl
from jax.experimental.pallas import tpu as pltpu
from jax.experimental.pallas import tpu_sc as plsc

_NC = 2
_NS = 16
_NW = _NC * _NS


def _pack2(x):
    h = x.shape[1] // 2
    lo = jax.lax.bitcast_convert_type(
        x[:, :h].astype(jnp.bfloat16), jnp.uint16).astype(jnp.uint32)
    hi = jax.lax.bitcast_convert_type(
        x[:, h:].astype(jnp.bfloat16), jnp.uint16).astype(jnp.uint32)
    return jax.lax.bitcast_convert_type(lo | (hi << 16), jnp.float32)


def _unpack2(x):
    u = jax.lax.bitcast_convert_type(x, jnp.uint32)
    lo = jax.lax.bitcast_convert_type(u << 16, jnp.float32)
    hi = jax.lax.bitcast_convert_type(u & jnp.uint32(0xFFFF0000), jnp.float32)
    return jnp.concatenate([lo, hi], axis=1)


def _proj_body(nf_ref, ws_ref, wd_ref, wv_ref, bv_ref, p_ref, q_ref, v_ref):
    n = nf_ref[...]
    p_ref[...] = _pack2(jnp.dot(n, ws_ref[...], preferred_element_type=jnp.float32))
    q_ref[...] = _pack2(jnp.dot(n, wd_ref[...], preferred_element_type=jnp.float32))
    v_ref[...] = jnp.dot(n, wv_ref[...], preferred_element_type=jnp.float32) + bv_ref[...]


def _node_proj(node_feats, Ws, Wd, Wv, bv, block=2000):
    n_nodes, in_node = node_feats.shape
    hidden = Ws.shape[1]
    out_node = Wv.shape[1]
    grid = (n_nodes // block,)
    return pl.pallas_call(
        _proj_body,
        grid=grid,
        in_specs=[
            pl.BlockSpec((block, in_node), lambda i: (i, 0)),
            pl.BlockSpec((in_node, hidden), lambda i: (0, 0)),
            pl.BlockSpec((in_node, hidden), lambda i: (0, 0)),
            pl.BlockSpec((in_node, out_node), lambda i: (0, 0)),
            pl.BlockSpec((1, out_node), lambda i: (0, 0)),
        ],
        out_specs=[
            pl.BlockSpec((block, hidden // 2), lambda i: (i, 0)),
            pl.BlockSpec((block, hidden // 2), lambda i: (i, 0)),
            pl.BlockSpec((block, out_node), lambda i: (i, 0)),
        ],
        out_shape=[
            jax.ShapeDtypeStruct((n_nodes, hidden // 2), jnp.float32),
            jax.ShapeDtypeStruct((n_nodes, hidden // 2), jnp.float32),
            jax.ShapeDtypeStruct((n_nodes, out_node), jnp.float32),
        ],
    )(node_feats, Ws, Wd, Wv, bv)


def _make_sc_gather(n_edges, hidden, chunk, dt):
    per_w = n_edges // _NW
    n_full = per_w // chunk
    tail = per_w % chunk
    assert tail % 8 == 0
    n_chunks = n_full + (1 if tail else 0)
    nbuf = 4
    n_main = max(0, (n_full - (nbuf - 1))) // nbuf
    n_static = n_chunks - nbuf * n_main
    mesh = plsc.VectorSubcoreMesh(core_axis_name="c", subcore_axis_name="s")

    def _size(c):
        return tail if (tail and c == n_chunks - 1) else chunk

    @functools.partial(
        pl.kernel,
        mesh=mesh,
        out_type=(
            jax.ShapeDtypeStruct((n_edges, hidden), dt),
            jax.ShapeDtypeStruct((n_edges, hidden), dt),
        ),
        scratch_types=[
            pltpu.VMEM((per_w,), jnp.int32),
            pltpu.VMEM((per_w,), jnp.int32),
        ] + [pltpu.VMEM((chunk, hidden), dt)] * (2 * nbuf)
          + [pltpu.SemaphoreType.DMA] * (4 * nbuf),
    )
    def gather2(p_hbm, q_hbm, src_hbm, dst_hbm, gp_hbm, gq_hbm, *scr):
        si_v, di_v = scr[:2]
        prs = scr[2:2 + nbuf]
        qrs = scr[2 + nbuf:2 + 2 * nbuf]
        sems = scr[2 + 2 * nbuf:]
        sgp, sgq = sems[:nbuf], sems[nbuf:2 * nbuf]
        swp, swq = sems[2 * nbuf:3 * nbuf], sems[3 * nbuf:]
        wid = lax.axis_index("s") * _NC + lax.axis_index("c")
        w_base = wid * per_w

        pltpu.sync_copy(src_hbm.at[pl.ds(w_base, per_w)], si_v)
        pltpu.sync_copy(dst_hbm.at[pl.ds(w_base, per_w)], di_v)

        def g_copies(c, b, sz=chunk):
            isl = pl.ds(c * chunk, sz)
            dsl = pl.ds(0, sz)
            return (pltpu.make_async_copy(p_hbm.at[si_v.at[isl]],
                                          prs[b].at[dsl], sgp[b]),
                    pltpu.make_async_copy(q_hbm.at[di_v.at[isl]],
                                          qrs[b].at[dsl], sgq[b]))

        def w_copies(c, b, sz=chunk):
            osl = pl.ds(w_base + c * chunk, sz)
            dsl = pl.ds(0, sz)
            return (pltpu.make_async_copy(prs[b].at[dsl], gp_hbm.at[osl], swp[b]),
                    pltpu.make_async_copy(qrs[b].at[dsl], gq_hbm.at[osl], swq[b]))

        def g_start(c, b, sz=chunk):
            for cp in g_copies(c, b, sz):
                cp.start()

        def process(c, b, static_tail):
            sz = _size(c) if static_tail is not None else chunk
            for cp in g_copies(c, b, sz):
                cp.wait()
            for cp in w_copies(c, b, sz):
                cp.start()
            pb = (b - 1) % nbuf
            if static_tail is None:
                @pl.when(jnp.logical_and(c >= 1, c + nbuf - 1 < n_chunks))
                def _():
                    for cp in w_copies(c - 1, pb):
                        cp.wait()
                    g_start(c + nbuf - 1, pb)
            elif static_tail:
                for cp in w_copies(c - 1, pb, _size(c - 1)):
                    cp.wait()
                g_start(c + nbuf - 1, pb, _size(c + nbuf - 1))

        for b in range(nbuf):
            g_start(b, b, _size(b))

        def body(g, carry):
            for k in range(nbuf):
                process(nbuf * g + k, k, None)
            return carry

        lax.fori_loop(0, n_main, body, 0)

        for c in range(nbuf * n_main, n_chunks):
            process(c, c % nbuf, c >= 1 and c + nbuf - 1 < n_chunks)
        for c in range(n_chunks - nbuf, n_chunks):
            for cp in w_copies(c, c % nbuf, _size(c)):
                cp.wait()

    return gather2


def _edge_body(gp_ref, gq_ref, eft_ref, w1e_ref, b1_ref, w2_ref, b2t_ref, out_ref):
    c = jax.lax.dot_general(eft_ref[...], w1e_ref[...], (((0,), (0,)), ((), ())),
                            preferred_element_type=jnp.float32)
    g = _unpack2(gp_ref[...]) + _unpack2(gq_ref[...])
    h = jnp.maximum(g + c + b1_ref[...], 0.0)
    out_ref[...] = jax.lax.dot_general(
        w2_ref[...], h, (((0,), (1,)), ((), ())),
        preferred_element_type=jnp.float32) + b2t_ref[...]


def _edge_mlp(Gp, Gq, edge_feats, W1e, b1, W2, b2, block=16000):
    n_edges, hidden_p = Gp.shape
    in_edge = edge_feats.shape[1]
    out_edge = W2.shape[1]
    eft = edge_feats.T
    grid = (n_edges // block,)
    e2t = pl.pallas_call(
        _edge_body,
        grid=grid,
        in_specs=[
            pl.BlockSpec((block, hidden_p), lambda i: (i, 0)),
            pl.BlockSpec((block, hidden_p), lambda i: (i, 0)),
            pl.BlockSpec((in_edge, block), lambda i: (0, i)),
            pl.BlockSpec((in_edge, W1e.shape[1]), lambda i: (0, 0)),
            pl.BlockSpec((1, W1e.shape[1]), lambda i: (0, 0)),
            pl.BlockSpec((W2.shape[0], out_edge), lambda i: (0, 0)),
            pl.BlockSpec((out_edge, 1), lambda i: (0, 0)),
        ],
        out_specs=pl.BlockSpec((out_edge, block), lambda i: (0, i)),
        out_shape=jax.ShapeDtypeStruct((out_edge, n_edges), jnp.float32),
    )(Gp, Gq, eft, W1e, b1, W2, b2.reshape(-1, 1))
    return e2t.T


def kernel(node_feats, edge_feats, edge_index, W1, b1, W2, b2, Wv, bv):
    in_node = node_feats.shape[1]
    in_edge = edge_feats.shape[1]
    n_edges = edge_feats.shape[0]
    hidden = W1.shape[1]

    Ws = W1[:in_node]
    We = W1[in_node:in_node + in_edge]
    Wd = W1[in_node + in_edge:]

    P, Q, V = _node_proj(node_feats, Ws, Wd, Wv, bv.reshape(1, -1))

    src_i = edge_index[0].astype(jnp.int32)
    dst_i = edge_index[1].astype(jnp.int32)
    Gp, Gq = _make_sc_gather(n_edges, hidden // 2, 96, jnp.float32)(
        P, Q, src_i, dst_i)
    e2 = _edge_mlp(Gp, Gq, edge_feats, We, b1.reshape(1, -1), W2,
                   b2.reshape(1, -1))
    return (V, e2)


# speedup vs baseline: 1.6064x; 1.0007x over previous
import functools

import jax
import jax.numpy as jnp
from jax import lax
from jax.experimental import pallas as pl
from jax.experimental.pallas import tpu as pltpu
from jax.experimental.pallas import tpu_sc as plsc

_NC = 2
_NS = 16
_NW = _NC * _NS


def _pack2(x):
    h = x.shape[1] // 2
    lo = jax.lax.bitcast_convert_type(
        x[:, :h].astype(jnp.bfloat16), jnp.uint16).astype(jnp.uint32)
    hi = jax.lax.bitcast_convert_type(
        x[:, h:].astype(jnp.bfloat16), jnp.uint16).astype(jnp.uint32)
    return jax.lax.bitcast_convert_type(lo | (hi << 16), jnp.float32)


def _unpack2(x):
    u = jax.lax.bitcast_convert_type(x, jnp.uint32)
    lo = jax.lax.bitcast_convert_type(u << 16, jnp.float32)
    hi = jax.lax.bitcast_convert_type(u & jnp.uint32(0xFFFF0000), jnp.float32)
    return jnp.concatenate([lo, hi], axis=1)


def _proj_body(nf_ref, ws_ref, wd_ref, wv_ref, bv_ref, p_ref, q_ref, v_ref):
    n = nf_ref[...]
    p_ref[...] = _pack2(jnp.dot(n, ws_ref[...], preferred_element_type=jnp.float32))
    q_ref[...] = _pack2(jnp.dot(n, wd_ref[...], preferred_element_type=jnp.float32))
    v_ref[...] = jnp.dot(n, wv_ref[...], preferred_element_type=jnp.float32) + bv_ref[...]


def _node_proj(node_feats, Ws, Wd, Wv, bv, block=2000):
    n_nodes, in_node = node_feats.shape
    hidden = Ws.shape[1]
    out_node = Wv.shape[1]
    grid = (n_nodes // block,)
    return pl.pallas_call(
        _proj_body,
        grid=grid,
        in_specs=[
            pl.BlockSpec((block, in_node), lambda i: (i, 0)),
            pl.BlockSpec((in_node, hidden), lambda i: (0, 0)),
            pl.BlockSpec((in_node, hidden), lambda i: (0, 0)),
            pl.BlockSpec((in_node, out_node), lambda i: (0, 0)),
            pl.BlockSpec((1, out_node), lambda i: (0, 0)),
        ],
        out_specs=[
            pl.BlockSpec((block, hidden // 2), lambda i: (i, 0)),
            pl.BlockSpec((block, hidden // 2), lambda i: (i, 0)),
            pl.BlockSpec((block, out_node), lambda i: (i, 0)),
        ],
        out_shape=[
            jax.ShapeDtypeStruct((n_nodes, hidden // 2), jnp.float32),
            jax.ShapeDtypeStruct((n_nodes, hidden // 2), jnp.float32),
            jax.ShapeDtypeStruct((n_nodes, out_node), jnp.float32),
        ],
    )(node_feats, Ws, Wd, Wv, bv)


def _make_sc_gather(n_edges, hidden, chunk, dt):
    per_w = n_edges // _NW
    n_full = per_w // chunk
    tail = per_w % chunk
    assert tail % 8 == 0
    n_chunks = n_full + (1 if tail else 0)
    nbuf = 4
    n_main = max(0, (n_full - (nbuf - 1))) // nbuf
    mesh = plsc.VectorSubcoreMesh(core_axis_name="c", subcore_axis_name="s")

    def _size(c):
        return tail if (tail and c == n_chunks - 1) else chunk

    @functools.partial(
        pl.kernel,
        mesh=mesh,
        out_type=(
            jax.ShapeDtypeStruct((n_edges, hidden), dt),
            jax.ShapeDtypeStruct((n_edges, hidden), dt),
        ),
        scratch_types=[
            pltpu.VMEM((per_w,), jnp.int32),
            pltpu.VMEM((per_w,), jnp.int32),
        ] + [pltpu.VMEM((chunk, hidden), dt)] * (2 * nbuf)
          + [pltpu.SemaphoreType.DMA] * (4 * nbuf),
    )
    def gather2(p_hbm, q_hbm, src_hbm, dst_hbm, gp_hbm, gq_hbm, *scr):
        si_v, di_v = scr[:2]
        prs = scr[2:2 + nbuf]
        qrs = scr[2 + nbuf:2 + 2 * nbuf]
        sems = scr[2 + 2 * nbuf:]
        sgp, sgq = sems[:nbuf], sems[nbuf:2 * nbuf]
        swp, swq = sems[2 * nbuf:3 * nbuf], sems[3 * nbuf:]
        wid = lax.axis_index("s") * _NC + lax.axis_index("c")
        w_base = wid * per_w

        pltpu.sync_copy(src_hbm.at[pl.ds(w_base, per_w)], si_v)
        pltpu.sync_copy(dst_hbm.at[pl.ds(w_base, per_w)], di_v)

        def g_copies(c, b, sz=chunk):
            isl = pl.ds(c * chunk, sz)
            dsl = pl.ds(0, sz)
            return (pltpu.make_async_copy(p_hbm.at[si_v.at[isl]],
                                          prs[b].at[dsl], sgp[b]),
                    pltpu.make_async_copy(q_hbm.at[di_v.at[isl]],
                                          qrs[b].at[dsl], sgq[b]))

        def w_copies(c, b, sz=chunk):
            osl = pl.ds(w_base + c * chunk, sz)
            dsl = pl.ds(0, sz)
            return (pltpu.make_async_copy(prs[b].at[dsl], gp_hbm.at[osl], swp[b]),
                    pltpu.make_async_copy(qrs[b].at[dsl], gq_hbm.at[osl], swq[b]))

        def g_start(c, b, sz=chunk):
            for cp in g_copies(c, b, sz):
                cp.start()

        def process(c, b, static_tail):
            sz = _size(c) if static_tail is not None else chunk
            for cp in g_copies(c, b, sz):
                cp.wait()
            for cp in w_copies(c, b, sz):
                cp.start()
            pb = (b - 1) % nbuf
            if static_tail is None:
                @pl.when(jnp.logical_and(c >= 1, c + nbuf - 1 < n_chunks))
                def _():
                    for cp in w_copies(c - 1, pb):
                        cp.wait()
                    g_start(c + nbuf - 1, pb)
            elif static_tail:
                for cp in w_copies(c - 1, pb, _size(c - 1)):
                    cp.wait()
                g_start(c + nbuf - 1, pb, _size(c + nbuf - 1))

        for b in range(nbuf):
            g_start(b, b, _size(b))

        def body(g, carry):
            for k in range(nbuf):
                process(nbuf * g + k, k, None)
            return carry

        lax.fori_loop(0, n_main, body, 0)

        for c in range(nbuf * n_main, n_chunks):
            process(c, c % nbuf, c >= 1 and c + nbuf - 1 < n_chunks)
        for c in range(n_chunks - nbuf, n_chunks):
            for cp in w_copies(c, c % nbuf, _size(c)):
                cp.wait()

    return gather2


def _edge_body(gp_ref, gq_ref, eft_ref, w1e_ref, b1_ref, w2_ref, b2t_ref, out_ref):
    c = jax.lax.dot_general(eft_ref[...], w1e_ref[...], (((0,), (0,)), ((), ())),
                            preferred_element_type=jnp.float32)
    g = _unpack2(gp_ref[...]) + _unpack2(gq_ref[...])
    h = jnp.maximum(g + c + b1_ref[...], 0.0)
    out_ref[...] = jax.lax.dot_general(
        w2_ref[...], h, (((0,), (1,)), ((), ())),
        preferred_element_type=jnp.float32) + b2t_ref[...]


def _edge_mlp(Gp, Gq, edge_feats, W1e, b1, W2, b2, block=16000):
    n_edges, hidden_p = Gp.shape
    in_edge = edge_feats.shape[1]
    out_edge = W2.shape[1]
    eft = edge_feats.T
    grid = (n_edges // block,)
    e2t = pl.pallas_call(
        _edge_body,
        grid=grid,
        in_specs=[
            pl.BlockSpec((block, hidden_p), lambda i: (i, 0)),
            pl.BlockSpec((block, hidden_p), lambda i: (i, 0)),
            pl.BlockSpec((in_edge, block), lambda i: (0, i)),
            pl.BlockSpec((in_edge, W1e.shape[1]), lambda i: (0, 0)),
            pl.BlockSpec((1, W1e.shape[1]), lambda i: (0, 0)),
            pl.BlockSpec((W2.shape[0], out_edge), lambda i: (0, 0)),
            pl.BlockSpec((out_edge, 1), lambda i: (0, 0)),
        ],
        out_specs=pl.BlockSpec((out_edge, block), lambda i: (0, i)),
        out_shape=jax.ShapeDtypeStruct((out_edge, n_edges), jnp.float32),
    )(Gp, Gq, eft, W1e, b1, W2, b2.reshape(-1, 1))
    return e2t.T


def kernel(node_feats, edge_feats, edge_index, W1, b1, W2, b2, Wv, bv):
    in_node = node_feats.shape[1]
    in_edge = edge_feats.shape[1]
    n_edges = edge_feats.shape[0]
    hidden = W1.shape[1]

    Ws = W1[:in_node]
    We = W1[in_node:in_node + in_edge]
    Wd = W1[in_node + in_edge:]

    P, Q, V = _node_proj(node_feats, Ws, Wd, Wv, bv.reshape(1, -1))

    src_i = edge_index[0].astype(jnp.int32)
    dst_i = edge_index[1].astype(jnp.int32)
    Gp, Gq = _make_sc_gather(n_edges, hidden // 2, 96, jnp.float32)(
        P, Q, src_i, dst_i)
    e2 = _edge_mlp(Gp, Gq, edge_feats, We, b1.reshape(1, -1), W2,
                   b2.reshape(1, -1))
    return (V, e2)
